# Initial kernel scaffold; baseline (speedup 1.0000x reference)
import jax
import jax.numpy as jnp
from jax.experimental import pallas as pl


def kernel(nodes, edges, globals_, senders, receivers, epsilon, W_e_kernel, W_e_bias, W1, b1, W2, b2):
    raise NotImplementedError("write your pallas kernel here")


# speedup vs baseline: 1.4677x; 1.4677x over previous
import jax
import jax.numpy as jnp
from jax import lax
from jax.experimental import pallas as pl
from jax.experimental.pallas import tpu as pltpu
from jax.experimental.pallas import tpu_sc as plsc

N, E, D, DE, DG, H = 10000, 160000, 256, 16, 128, 512

NC, NS = 2, 16
NW = NC * NS
EP = 163840
EPAD = EP - E

G_PER_W = EP // NW
G_IDX = 128
G_CHUNK = 256
G_OPS = G_CHUNK // G_IDX
G_NCH = G_PER_W // G_CHUNK
G_ROWS = G_PER_W // G_IDX

DH = D // NC
S_PER_T = EP // NS
S_IDX = 128
S_CHUNK = 512
S_OPS = S_CHUNK // S_IDX
S_NCH = S_PER_T // S_CHUNK
S_ROWS = S_PER_T // S_IDX
HALF = N // 2
TRASH = 128
ACC_R = HALF + TRASH
ZR = 320
WR = 312

RB_E = 2048
RB_N = 1000


def _sc_gather_body(idx_hbm, table_hbm, out_hbm, idx_v, buf, sem):
    c = lax.axis_index("c")
    s = lax.axis_index("s")
    w = s * NC + c
    pltpu.sync_copy(idx_hbm.at[w], idx_v)

    def chunk(i, _):
        handles = []
        for j in range(G_OPS):
            handles.append(pltpu.async_copy(
                table_hbm.at[idx_v.at[i * G_OPS + j]],
                buf.at[pl.ds(j * G_IDX, G_IDX), :], sem))
        for h in handles:
            h.wait()
        base = w * G_PER_W + i * G_CHUNK
        pltpu.sync_copy(buf, out_hbm.at[pl.ds(base, G_CHUNK), :])
        return 0

    lax.fori_loop(0, G_NCH, chunk, 0)


def _sc_scatter_body(ridx_hbm, e_hbm, out_hbm, idx_raw, idx_v, buf, acc):
    c = lax.axis_index("c")
    s = lax.axis_index("s")
    zero16 = jnp.zeros((16,), jnp.float32)

    pltpu.sync_copy(ridx_hbm.at[s], idx_raw)

    for q in (0, 1):
        def zrow(r, _):
            for k in range(DH // 16):
                buf[r, pl.ds(k * 16, 16)] = zero16
            return 0

        lax.fori_loop(0, ZR, zrow, 0)
        pltpu.sync_copy(buf.at[pl.ds(0, ZR), :], acc.at[pl.ds(s * ZR, ZR), :])

        @pl.when(s == NS - 1)
        def _zero_tail():
            pltpu.sync_copy(buf.at[pl.ds(0, ACC_R - NS * ZR), :],
                            acc.at[pl.ds(NS * ZR, ACC_R - NS * ZR), :])

        lo = q * HALF

        def remap(g, _):
            for k in range(S_IDX // 16):
                r = idx_raw[g, pl.ds(k * 16, 16)]
                local = r - lo
                in_half = (r >= lo) & (r < lo + HALF)
                trash = HALF + (r & (TRASH - 1))
                idx_v[g, pl.ds(k * 16, 16)] = jnp.where(in_half, local, trash)
            return 0

        lax.fori_loop(0, S_ROWS, remap, 0)
        plsc.subcore_barrier()

        def chunk(i, _):
            row0 = s * S_PER_T + i * S_CHUNK
            pltpu.sync_copy(e_hbm.at[pl.ds(row0, S_CHUNK), pl.ds(c * DH, DH)], buf)
            for j in range(S_OPS):
                pltpu.sync_copy(buf.at[pl.ds(j * S_IDX, S_IDX), :],
                                acc.at[idx_v.at[i * S_OPS + j]], add=True)
            return 0

        lax.fori_loop(0, S_NCH, chunk, 0)
        plsc.subcore_barrier()

        pltpu.sync_copy(acc.at[pl.ds(s * WR, WR), :],
                        out_hbm.at[pl.ds(lo + s * WR, WR), pl.ds(c * DH, DH)])

        @pl.when(s == NS - 1)
        def _write_tail():
            pltpu.sync_copy(acc.at[pl.ds(NS * WR, HALF - NS * WR), :],
                            out_hbm.at[pl.ds(lo + NS * WR, HALF - NS * WR),
                                       pl.ds(c * DH, DH)])

        plsc.subcore_barrier()


_gather_call = pl.kernel(
    _sc_gather_body,
    out_type=jax.ShapeDtypeStruct((EP, D), jnp.float32),
    mesh=plsc.VectorSubcoreMesh(core_axis_name="c", subcore_axis_name="s"),
    scratch_types=[
        pltpu.VMEM((G_ROWS, G_IDX), jnp.int32),
        pltpu.VMEM((G_CHUNK, D), jnp.float32),
        pltpu.SemaphoreType.DMA,
    ],
)

_scatter_call = pl.kernel(
    _sc_scatter_body,
    out_type=jax.ShapeDtypeStruct((N, D), jnp.float32),
    mesh=plsc.VectorSubcoreMesh(core_axis_name="c", subcore_axis_name="s"),
    scratch_types=[
        pltpu.VMEM((S_ROWS, S_IDX), jnp.int32),
        pltpu.VMEM((S_ROWS, S_IDX), jnp.int32),
        pltpu.VMEM((S_CHUNK, DH), jnp.float32),
        pltpu.VMEM_SHARED((ACC_R, DH), jnp.float32),
    ],
)


def _edge_tc(sent_ref, edges_ref, we_ref, be_ref, out_ref):
    z = jnp.dot(edges_ref[...], we_ref[...], preferred_element_type=jnp.float32)
    x = sent_ref[...] + z + be_ref[...]
    sp = jnp.log(1.0 + jnp.exp(-jnp.abs(x))) + jnp.maximum(x, 0.0)
    out_ref[...] = x * jnp.tanh(sp)


def _mlp_tc(nodes_ref, recv_ref, g_ref, eps_ref, w1a_ref, w1b_ref, b1_ref,
            w2_ref, b2_ref, out_ref):
    h = (1.0 + eps_ref[...]) * nodes_ref[...] + recv_ref[...]
    gv = jnp.dot(g_ref[...], w1b_ref[...], preferred_element_type=jnp.float32) + b1_ref[...]
    t = jnp.maximum(jnp.dot(h, w1a_ref[...], preferred_element_type=jnp.float32) + gv, 0.0)
    out_ref[...] = jnp.dot(t, w2_ref[...], preferred_element_type=jnp.float32) + b2_ref[...]


def kernel(nodes, edges, globals_, senders, receivers, epsilon,
           W_e_kernel, W_e_bias, W1, b1, W2, b2):
    senders_p = jnp.concatenate(
        [senders, jnp.zeros((EPAD,), jnp.int32)]).reshape(NW, G_ROWS, G_IDX)
    sent = _gather_call(senders_p, nodes)

    edges_p = jnp.concatenate([edges, jnp.zeros((EPAD, DE), jnp.float32)])
    e = pl.pallas_call(
        _edge_tc,
        grid=(EP // RB_E,),
        in_specs=[
            pl.BlockSpec((RB_E, D), lambda i: (i, 0)),
            pl.BlockSpec((RB_E, DE), lambda i: (i, 0)),
            pl.BlockSpec((DE, D), lambda i: (0, 0)),
            pl.BlockSpec((1, D), lambda i: (0, 0)),
        ],
        out_specs=pl.BlockSpec((RB_E, D), lambda i: (i, 0)),
        out_shape=jax.ShapeDtypeStruct((EP, D), jnp.float32),
    )(sent, edges_p, W_e_kernel, W_e_bias.reshape(1, D))

    receivers_p = jnp.concatenate(
        [receivers, jnp.full((EPAD,), N, jnp.int32)]).reshape(NS, S_ROWS, S_IDX)
    recv = _scatter_call(receivers_p, e)

    out = pl.pallas_call(
        _mlp_tc,
        grid=(N // RB_N,),
        in_specs=[
            pl.BlockSpec((RB_N, D), lambda i: (i, 0)),
            pl.BlockSpec((RB_N, D), lambda i: (i, 0)),
            pl.BlockSpec((1, DG), lambda i: (0, 0)),
            pl.BlockSpec((1, 1), lambda i: (0, 0)),
            pl.BlockSpec((D, H), lambda i: (0, 0)),
            pl.BlockSpec((DG, H), lambda i: (0, 0)),
            pl.BlockSpec((1, H), lambda i: (0, 0)),
            pl.BlockSpec((H, D), lambda i: (0, 0)),
            pl.BlockSpec((1, D), lambda i: (0, 0)),
        ],
        out_specs=pl.BlockSpec((RB_N, D), lambda i: (i, 0)),
        out_shape=jax.ShapeDtypeStruct((N, D), jnp.float32),
    )(nodes, recv, globals_, epsilon, W1[:D], W1[D:], b1.reshape(1, H),
      W2, b2.reshape(1, D))
    return out


# speedup vs baseline: 1.6914x; 1.1525x over previous
import jax
import jax.numpy as jnp
from jax import lax
from jax.experimental import pallas as pl
from jax.experimental.pallas import tpu as pltpu
from jax.experimental.pallas import tpu_sc as plsc

N, E, D, DE, DG, H = 10000, 160000, 256, 16, 128, 512

NC, NS = 2, 16
NW = NC * NS
EP = 163840
EPAD = EP - E

G_PER_W = EP // NW
G_IDX = 128
G_CHUNK = 256
G_OPS = G_CHUNK // G_IDX
G_NCH = G_PER_W // G_CHUNK
G_ROWS = G_PER_W // G_IDX

DH = D // NC
S_PER_T = EP // NS
S_IDX = 128
S_CHUNK = 256
S_OPS = S_CHUNK // S_IDX
S_NCH = S_PER_T // S_CHUNK
S_ROWS = S_PER_T // S_IDX
TRASH = 8
ACC_R = N + TRASH
ZR = 632
ZR_LAST = ACC_R - 15 * ZR
WR = 624
WR_TAIL = N - NS * WR

RB_E = 2048
RB_N = 1000


def _sc_gather_body(idx_hbm, table_hbm, out_hbm, idx_v, buf, sem):
    c = lax.axis_index("c")
    s = lax.axis_index("s")
    w = s * NC + c
    pltpu.sync_copy(idx_hbm.at[w], idx_v)

    def chunk(i, _):
        handles = []
        for j in range(G_OPS):
            handles.append(pltpu.async_copy(
                table_hbm.at[idx_v.at[i * G_OPS + j]],
                buf.at[pl.ds(j * G_IDX, G_IDX), :], sem))
        for h in handles:
            h.wait()
        base = w * G_PER_W + i * G_CHUNK
        pltpu.sync_copy(buf, out_hbm.at[pl.ds(base, G_CHUNK), :])
        return 0

    lax.fori_loop(0, G_NCH, chunk, 0)


def _sc_scatter_body(ridx_hbm, e_hbm, out_hbm, idx_v, buf, acc):
    c = lax.axis_index("c")
    s = lax.axis_index("s")
    zero16 = jnp.zeros((16,), jnp.float32)

    def zrow(r, _):
        for k in range(DH // 16):
            buf[r, pl.ds(k * 16, 16)] = zero16
        return 0

    lax.fori_loop(0, S_CHUNK, zrow, 0)

    @pl.when(s < NS - 1)
    def _zero_main():
        base = s * ZR
        pltpu.sync_copy(buf, acc.at[pl.ds(base, S_CHUNK), :])
        pltpu.sync_copy(buf, acc.at[pl.ds(base + S_CHUNK, S_CHUNK), :])
        pltpu.sync_copy(buf.at[pl.ds(0, ZR - 2 * S_CHUNK), :],
                        acc.at[pl.ds(base + 2 * S_CHUNK, ZR - 2 * S_CHUNK), :])

    @pl.when(s == NS - 1)
    def _zero_last():
        base = (NS - 1) * ZR
        pltpu.sync_copy(buf, acc.at[pl.ds(base, S_CHUNK), :])
        pltpu.sync_copy(buf, acc.at[pl.ds(base + S_CHUNK, S_CHUNK), :])
        pltpu.sync_copy(buf.at[pl.ds(0, ZR_LAST - 2 * S_CHUNK), :],
                        acc.at[pl.ds(base + 2 * S_CHUNK, ZR_LAST - 2 * S_CHUNK), :])

    pltpu.sync_copy(ridx_hbm.at[s], idx_v)
    plsc.subcore_barrier()

    def chunk(i, _):
        row0 = s * S_PER_T + i * S_CHUNK
        pltpu.sync_copy(e_hbm.at[pl.ds(row0, S_CHUNK), pl.ds(c * DH, DH)], buf)
        for j in range(S_OPS):
            pltpu.sync_copy(buf.at[pl.ds(j * S_IDX, S_IDX), :],
                            acc.at[idx_v.at[i * S_OPS + j]], add=True)
        return 0

    lax.fori_loop(0, S_NCH, chunk, 0)
    plsc.subcore_barrier()

    pltpu.sync_copy(acc.at[pl.ds(s * WR, WR), :],
                    out_hbm.at[pl.ds(s * WR, WR), pl.ds(c * DH, DH)])

    @pl.when(s == NS - 1)
    def _write_tail():
        pltpu.sync_copy(acc.at[pl.ds(NS * WR, WR_TAIL), :],
                        out_hbm.at[pl.ds(NS * WR, WR_TAIL), pl.ds(c * DH, DH)])


_gather_call = pl.kernel(
    _sc_gather_body,
    out_type=jax.ShapeDtypeStruct((EP, D), jnp.float32),
    mesh=plsc.VectorSubcoreMesh(core_axis_name="c", subcore_axis_name="s"),
    scratch_types=[
        pltpu.VMEM((G_ROWS, G_IDX), jnp.int32),
        pltpu.VMEM((G_CHUNK, D), jnp.float32),
        pltpu.SemaphoreType.DMA,
    ],
)

_scatter_call = pl.kernel(
    _sc_scatter_body,
    out_type=jax.ShapeDtypeStruct((N, D), jnp.float32),
    mesh=plsc.VectorSubcoreMesh(core_axis_name="c", subcore_axis_name="s"),
    scratch_types=[
        pltpu.VMEM((S_ROWS, S_IDX), jnp.int32),
        pltpu.VMEM((S_CHUNK, DH), jnp.float32),
        pltpu.VMEM_SHARED((ACC_R, DH), jnp.float32),
    ],
)


def _edge_tc(sent_ref, edges_ref, we_ref, be_ref, out_ref):
    z = jnp.dot(edges_ref[...], we_ref[...], preferred_element_type=jnp.float32)
    x = sent_ref[...] + z + be_ref[...]
    sp = jnp.log(1.0 + jnp.exp(-jnp.abs(x))) + jnp.maximum(x, 0.0)
    out_ref[...] = x * jnp.tanh(sp)


def _mlp_tc(nodes_ref, recv_ref, g_ref, eps_ref, w1a_ref, w1b_ref, b1_ref,
            w2_ref, b2_ref, out_ref):
    h = (1.0 + eps_ref[...]) * nodes_ref[...] + recv_ref[...]
    gv = jnp.dot(g_ref[...], w1b_ref[...], preferred_element_type=jnp.float32) + b1_ref[...]
    t = jnp.maximum(jnp.dot(h, w1a_ref[...], preferred_element_type=jnp.float32) + gv, 0.0)
    out_ref[...] = jnp.dot(t, w2_ref[...], preferred_element_type=jnp.float32) + b2_ref[...]


def kernel(nodes, edges, globals_, senders, receivers, epsilon,
           W_e_kernel, W_e_bias, W1, b1, W2, b2):
    senders_p = jnp.concatenate(
        [senders, jnp.zeros((EPAD,), jnp.int32)]).reshape(NW, G_ROWS, G_IDX)
    sent = _gather_call(senders_p, nodes)

    edges_p = jnp.concatenate([edges, jnp.zeros((EPAD, DE), jnp.float32)])
    e = pl.pallas_call(
        _edge_tc,
        grid=(EP // RB_E,),
        in_specs=[
            pl.BlockSpec((RB_E, D), lambda i: (i, 0)),
            pl.BlockSpec((RB_E, DE), lambda i: (i, 0)),
            pl.BlockSpec((DE, D), lambda i: (0, 0)),
            pl.BlockSpec((1, D), lambda i: (0, 0)),
        ],
        out_specs=pl.BlockSpec((RB_E, D), lambda i: (i, 0)),
        out_shape=jax.ShapeDtypeStruct((EP, D), jnp.float32),
    )(sent, edges_p, W_e_kernel, W_e_bias.reshape(1, D))

    pad_ids = N + (jnp.arange(EPAD, dtype=jnp.int32) % TRASH)
    receivers_p = jnp.concatenate(
        [receivers, pad_ids]).reshape(NS, S_ROWS, S_IDX)
    recv = _scatter_call(receivers_p, e)

    out = pl.pallas_call(
        _mlp_tc,
        grid=(N // RB_N,),
        in_specs=[
            pl.BlockSpec((RB_N, D), lambda i: (i, 0)),
            pl.BlockSpec((RB_N, D), lambda i: (i, 0)),
            pl.BlockSpec((1, DG), lambda i: (0, 0)),
            pl.BlockSpec((1, 1), lambda i: (0, 0)),
            pl.BlockSpec((D, H), lambda i: (0, 0)),
            pl.BlockSpec((DG, H), lambda i: (0, 0)),
            pl.BlockSpec((1, H), lambda i: (0, 0)),
            pl.BlockSpec((H, D), lambda i: (0, 0)),
            pl.BlockSpec((1, D), lambda i: (0, 0)),
        ],
        out_specs=pl.BlockSpec((RB_N, D), lambda i: (i, 0)),
        out_shape=jax.ShapeDtypeStruct((N, D), jnp.float32),
    )(nodes, recv, globals_, epsilon, W1[:D], W1[D:], b1.reshape(1, H),
      W2, b2.reshape(1, D))
    return out
